# baseline (speedup 1.0000x reference)
import functools

import jax
import jax.numpy as jnp
from jax import lax
from jax.experimental import pallas as pl
from jax.experimental.pallas import tpu as pltpu
from jax.experimental.pallas import tpu_sc as plsc

C = 192
H = 224
W = 224
OH = 7
OW = 7
NB = OH * OW
N_ROIS = 1000

NW = 32
R_PER_W = 32
L = 16
NCH = C // L
NG = 4
RSTRIDE = 208
WSTRIDE = 256

_mesh = plsc.VectorSubcoreMesh(core_axis_name="c", subcore_axis_name="s")


@functools.partial(
    pl.kernel,
    out_type=jax.ShapeDtypeStruct((N_ROIS, C * NB), jnp.float32),
    mesh=_mesh,
    compiler_params=pltpu.CompilerParams(
        needs_layout_passes=False, use_tc_tiling_on_sc=False),
    scratch_types=[
        pltpu.VMEM((4 * R_PER_W,), jnp.float32),
        pltpu.VMEM((64,), jnp.float32),
        pltpu.VMEM((64,), jnp.float32),
        pltpu.VMEM((R_PER_W * RSTRIDE,), jnp.int32),
        pltpu.VMEM((R_PER_W * WSTRIDE,), jnp.float32),
        pltpu.VMEM((4 * NB, C), jnp.float32),
        pltpu.VMEM((4 * NB, C), jnp.float32),
        pltpu.VMEM((C * NB,), jnp.float32),
        pltpu.VMEM((C * NB,), jnp.float32),
        pltpu.VMEM((C * NB,), jnp.float32),
        pltpu.SMEM((R_PER_W,), jnp.int32),
        pltpu.SMEM((R_PER_W,), jnp.int32),
        pltpu.SMEM((2,), jnp.int32),
        pltpu.SemaphoreType.DMA,
        pltpu.SemaphoreType.DMA,
        pltpu.SemaphoreType.DMA,
        pltpu.SemaphoreType.DMA,
        pltpu.SemaphoreType.DMA,
    ],
)
def _roi_align_sc(feat_hbm, rois_hbm, ohp5_hbm, owp5_hbm, zeros_hbm, out_hbm,
                  rois_v, ohp5_v, owp5_v, idx_v, w_v, g0, g1, ob0, ob1, zb,
                  f_valid, f_zero, pend, sem0, sem1, semz, semo0, semo1):
    wid = lax.axis_index("s") * 2 + lax.axis_index("c")
    n_base = wid * R_PER_W

    pltpu.sync_copy(rois_hbm.at[pl.ds(n_base * 4, 4 * R_PER_W)], rois_v)
    pltpu.sync_copy(ohp5_hbm, ohp5_v)
    pltpu.sync_copy(owp5_hbm, owp5_v)
    pltpu.sync_copy(zeros_hbm, zb)

    iota16 = lax.iota(jnp.int32, 16)
    zeros16 = jnp.zeros((L,), jnp.int32)
    tail_mask = iota16 < 1

    def flag_body(r, acc):
        n = n_base + r
        r4 = zeros16 + r * 4
        x1 = plsc.load_gather(rois_v, [r4]) * float(W)
        y1 = plsc.load_gather(rois_v, [r4 + 1]) * float(H)
        x2 = plsc.load_gather(rois_v, [r4 + 2]) * float(W)
        y2 = plsc.load_gather(rois_v, [r4 + 3]) * float(H)
        bh = (y2 - y1) / float(OH)
        bw = (x2 - x1) / float(OW)
        validv = jnp.where((bh > 0.0) & (bw > 0.0), 1, 0)
        vs = jnp.max(validv)
        exists = jnp.where(n < N_ROIS, 1, 0)
        zflag = (1 - vs) * exists
        f_valid[r] = vs * exists
        f_zero[r] = zflag
        for g in range(NG):
            ohv = ohp5_v[pl.ds(g * L, L)]
            owv = owp5_v[pl.ds(g * L, L)]
            y = jnp.clip(y1 + ohv * bh, 0.0, float(H - 1))
            x = jnp.clip(x1 + owv * bw, 0.0, float(W - 1))
            y0 = y.astype(jnp.int32)
            x0 = x.astype(jnp.int32)
            wy = y - y0.astype(jnp.float32)
            wx = x - x0.astype(jnp.float32)
            y1i = jnp.minimum(y0 + 1, H - 1)
            x1i = jnp.minimum(x0 + 1, W - 1)
            omwx = 1.0 - wx
            omwy = 1.0 - wy
            yb0 = y0 * W
            yb1 = y1i * W
            corners = ((yb0 + x0, omwx * omwy), (yb0 + x1i, wx * omwy),
                       (yb1 + x0, omwx * wy), (yb1 + x1i, wx * wy))
            for c, (pc, wc) in enumerate(corners):
                pos = iota16 + (r * RSTRIDE + c * NB + g * L)
                if g == NG - 1:
                    plsc.store_scatter(idx_v, [pos], pc, mask=tail_mask)
                else:
                    plsc.store_scatter(idx_v, [pos], pc)
                w_v[pl.ds(r * WSTRIDE + c * 64 + g * L, L)] = wc
        return acc + zflag

    zcnt = lax.fori_loop(0, R_PER_W, flag_body, 0)

    def start_gather(r, gbuf, sem):
        ib = r * RSTRIDE
        pltpu.async_copy(feat_hbm.at[idx_v.at[pl.ds(ib, 128)]],
                         gbuf.at[pl.ds(0, 128)], sem)
        pltpu.async_copy(feat_hbm.at[idx_v.at[pl.ds(ib + 128, 68)]],
                         gbuf.at[pl.ds(128, 68)], sem)

    def wait_gather(r, gbuf, sem):
        ib = r * RSTRIDE
        pltpu.make_async_copy(feat_hbm.at[idx_v.at[pl.ds(ib, 128)]],
                              gbuf.at[pl.ds(0, 128)], sem).wait()
        pltpu.make_async_copy(feat_hbm.at[idx_v.at[pl.ds(ib + 128, 68)]],
                              gbuf.at[pl.ds(128, 68)], sem).wait()

    def combine(r, gbuf, ob):
        wb = r * WSTRIDE

        def samp_body(s, _):
            w00 = plsc.load_gather(w_v, [zeros16 + (wb + 0 * 64) + s])
            w01 = plsc.load_gather(w_v, [zeros16 + (wb + 1 * 64) + s])
            w10 = plsc.load_gather(w_v, [zeros16 + (wb + 2 * 64) + s])
            w11 = plsc.load_gather(w_v, [zeros16 + (wb + 3 * 64) + s])
            sbase = s * C
            for j in range(NCH):
                cs = pl.ds(j * L, L)
                acc = (gbuf[0 * NB + s, cs] * w00
                       + gbuf[1 * NB + s, cs] * w01
                       + gbuf[2 * NB + s, cs] * w10
                       + gbuf[3 * NB + s, cs] * w11)
                ob[pl.ds(sbase + j * L, L)] = acc
            return 0

        lax.fori_loop(0, NB, samp_body, 0)

    gbufs = (g0, g1)
    sems = (sem0, sem1)
    obufs = (ob0, ob1)
    osems = (semo0, semo1)
    pend[0] = 0
    pend[1] = 0

    @pl.when(f_valid[0] == 1)
    def _():
        start_gather(0, g0, sem0)

    def pipe_body(rr, _):
        for b in range(2):
            r = rr * 2 + b
            nxt = r + 1
            prefetch = jnp.logical_and(nxt < R_PER_W, f_valid[nxt] == 1)

            @pl.when(prefetch)
            def _():
                start_gather(nxt, gbufs[1 - b], sems[1 - b])

            @pl.when(f_valid[r] == 1)
            def _():
                wait_gather(r, gbufs[b], sems[b])

                @pl.when(pend[b] == 1)
                def _():
                    pltpu.make_async_copy(
                        obufs[b], out_hbm.at[n_base], osems[b]).wait()

                combine(r, gbufs[b], obufs[b])
                pltpu.async_copy(obufs[b], out_hbm.at[n_base + r], osems[b])
                pend[b] = 1

            @pl.when(f_zero[r] == 1)
            def _():
                pltpu.async_copy(zb, out_hbm.at[n_base + r], semz)
        return 0

    lax.fori_loop(0, R_PER_W // 2, pipe_body, 0)

    for b in range(2):
        @pl.when(pend[b] == 1)
        def _():
            pltpu.make_async_copy(
                obufs[b], out_hbm.at[n_base], osems[b]).wait()

    def zdrain(i, _):
        pltpu.make_async_copy(zeros_hbm, zb, semz).wait()
        return 0

    lax.fori_loop(0, zcnt, zdrain, 0)


def kernel(features, rois):
    feat_t = jnp.transpose(features, (1, 2, 0)).reshape(H * W, C)
    rois_pad = jnp.pad(rois.reshape(-1), (0, 4 * (NW * R_PER_W - N_ROIS)))
    s = jnp.arange(64, dtype=jnp.int32)
    ohp5 = jnp.minimum(s // OW, OH - 1).astype(jnp.float32) + 0.5
    owp5 = (s % OW).astype(jnp.float32) + 0.5
    zeros_tile = jnp.zeros((C * NB,), jnp.float32)
    out = _roi_align_sc(feat_t, rois_pad, ohp5, owp5, zeros_tile)
    return out.reshape(N_ROIS, OH, OW, C).transpose(0, 3, 1, 2)
